# Initial kernel scaffold; baseline (speedup 1.0000x reference)
import functools

import jax
import jax.numpy as jnp
from jax import lax
from jax.experimental import pallas as pl
from jax.experimental.pallas import tpu as pltpu
from jax.experimental.pallas import tpu_sc as plsc

NUM_NODES = 1000000
DIM = 32

NC = 2
NS = 16
NW = NC * NS
ROWS_PER_W = NUM_NODES // NW
CHUNK = 1250
NCHUNK = ROWS_PER_W // CHUNK
NBUF = 2


@functools.partial(
    pl.kernel,
    out_type=jax.ShapeDtypeStruct((NUM_NODES, DIM), jnp.float32),
    mesh=plsc.VectorSubcoreMesh(core_axis_name="c", subcore_axis_name="s"),
    scratch_types=[
        pltpu.VMEM((NBUF, CHUNK, DIM), jnp.float32),
        pltpu.SemaphoreType.DMA,
        pltpu.SemaphoreType.DMA,
        pltpu.SemaphoreType.DMA,
        pltpu.SemaphoreType.DMA,
    ],
)
def _copy_table(w_hbm, out_hbm, buf, sem_in0, sem_in1, sem_out0, sem_out1):
    wid = lax.axis_index("s") * NC + lax.axis_index("c")
    base = wid * ROWS_PER_W
    sem_in = (sem_in0, sem_in1)
    sem_out = (sem_out0, sem_out1)

    def in_copy(g):
        b = g % NBUF
        return pltpu.make_async_copy(
            w_hbm.at[pl.ds(base + g * CHUNK, CHUNK), :], buf.at[b], sem_in[b]
        )

    def out_copy(g):
        b = g % NBUF
        return pltpu.make_async_copy(
            buf.at[b], out_hbm.at[pl.ds(base + g * CHUNK, CHUNK), :], sem_out[b]
        )

    in_copy(0).start()
    in_copy(1).start()
    for g in range(NCHUNK):
        in_copy(g).wait()
        out_copy(g).start()
        if g + 2 < NCHUNK:
            out_copy(g).wait()
            in_copy(g + 2).start()
    out_copy(NCHUNK - 2).wait()
    out_copy(NCHUNK - 1).wait()


def kernel(W):
    return _copy_table(W)


# speedup vs baseline: 1.4425x; 1.4425x over previous
import functools

import jax
import jax.numpy as jnp
from jax import lax
from jax.experimental import pallas as pl
from jax.experimental.pallas import tpu as pltpu
from jax.experimental.pallas import tpu_sc as plsc

NUM_NODES = 1000000
DIM = 32

NC = 2
NS = 16
NW = NC * NS
ROWS_PER_W = 31248
CHUNK = 1736
NCHUNK = ROWS_PER_W // CHUNK
NBUF = 2
TAIL_BASE = ROWS_PER_W * NW
TAIL = NUM_NODES - TAIL_BASE


@functools.partial(
    pl.kernel,
    out_type=jax.ShapeDtypeStruct((NUM_NODES, DIM), jnp.float32),
    mesh=plsc.VectorSubcoreMesh(core_axis_name="c", subcore_axis_name="s"),
    scratch_types=[
        pltpu.VMEM((NBUF, CHUNK, DIM), jnp.float32),
        pltpu.SemaphoreType.DMA,
        pltpu.SemaphoreType.DMA,
        pltpu.SemaphoreType.DMA,
        pltpu.SemaphoreType.DMA,
    ],
    compiler_params=pltpu.CompilerParams(use_tc_tiling_on_sc=False),
)
def _copy_table(w_hbm, out_hbm, buf, sem_in0, sem_in1, sem_out0, sem_out1):
    wid = lax.axis_index("s") * NC + lax.axis_index("c")
    base = pl.multiple_of(wid * ROWS_PER_W, 8)
    sem_in = (sem_in0, sem_in1)
    sem_out = (sem_out0, sem_out1)

    def in_copy(g):
        b = g % NBUF
        return pltpu.make_async_copy(
            w_hbm.at[pl.ds(base + g * CHUNK, CHUNK), :], buf.at[b], sem_in[b]
        )

    def out_copy(g):
        b = g % NBUF
        return pltpu.make_async_copy(
            buf.at[b], out_hbm.at[pl.ds(base + g * CHUNK, CHUNK), :], sem_out[b]
        )

    in_copy(0).start()
    in_copy(1).start()
    for g in range(NCHUNK):
        in_copy(g).wait()
        out_copy(g).start()
        if g + 2 < NCHUNK:
            out_copy(g).wait()
            in_copy(g + 2).start()
    out_copy(NCHUNK - 2).wait()
    out_copy(NCHUNK - 1).wait()

    @pl.when(wid == 0)
    def _tail():
        pltpu.sync_copy(
            w_hbm.at[pl.ds(TAIL_BASE, TAIL), :], buf.at[0, pl.ds(0, TAIL), :]
        )
        pltpu.sync_copy(
            buf.at[0, pl.ds(0, TAIL), :], out_hbm.at[pl.ds(TAIL_BASE, TAIL), :]
        )


def kernel(W):
    return _copy_table(W)


# speedup vs baseline: 1.6628x; 1.1527x over previous
import functools

import jax
import jax.numpy as jnp
from jax import lax
from jax.experimental import pallas as pl
from jax.experimental.pallas import tpu as pltpu
from jax.experimental.pallas import tpu_sc as plsc

NUM_NODES = 1000000
DIM = 32

NC = 2
NS = 16
NW = NC * NS
ROWS_PER_W = 31248
CHUNK = 504
NCHUNK = ROWS_PER_W // CHUNK
NBUF = 2
TAIL_BASE = ROWS_PER_W * NW
TAIL = NUM_NODES - TAIL_BASE


@functools.partial(
    pl.kernel,
    out_type=jax.ShapeDtypeStruct((NUM_NODES, DIM), jnp.float32),
    mesh=plsc.VectorSubcoreMesh(core_axis_name="c", subcore_axis_name="s"),
    scratch_types=[
        pltpu.VMEM((NBUF, CHUNK, DIM), jnp.float32),
        pltpu.SemaphoreType.DMA,
        pltpu.SemaphoreType.DMA,
        pltpu.SemaphoreType.DMA,
        pltpu.SemaphoreType.DMA,
    ],
)
def _copy_table(w_hbm, out_hbm, buf, sem_in0, sem_in1, sem_out0, sem_out1):
    wid = lax.axis_index("s") * NC + lax.axis_index("c")
    base = pl.multiple_of(wid * ROWS_PER_W, 8)
    sem_in = (sem_in0, sem_in1)
    sem_out = (sem_out0, sem_out1)

    def in_copy(g):
        b = g % NBUF
        return pltpu.make_async_copy(
            w_hbm.at[pl.ds(base + g * CHUNK, CHUNK), :], buf.at[b], sem_in[b]
        )

    def out_copy(g):
        b = g % NBUF
        return pltpu.make_async_copy(
            buf.at[b], out_hbm.at[pl.ds(base + g * CHUNK, CHUNK), :], sem_out[b]
        )

    in_copy(0).start()
    in_copy(1).start()
    for g in range(NCHUNK):
        in_copy(g).wait()
        out_copy(g).start()
        if g + 2 < NCHUNK:
            out_copy(g).wait()
            in_copy(g + 2).start()
    out_copy(NCHUNK - 2).wait()
    out_copy(NCHUNK - 1).wait()

    @pl.when(wid == 0)
    def _tail():
        pltpu.sync_copy(
            w_hbm.at[pl.ds(TAIL_BASE, TAIL), :], buf.at[0, pl.ds(0, TAIL), :]
        )
        pltpu.sync_copy(
            buf.at[0, pl.ds(0, TAIL), :], out_hbm.at[pl.ds(TAIL_BASE, TAIL), :]
        )


def kernel(W):
    return _copy_table(W)


# speedup vs baseline: 14.3937x; 8.6562x over previous
import functools

import jax
import jax.numpy as jnp
from jax import lax
from jax.experimental import pallas as pl
from jax.experimental.pallas import tpu as pltpu
from jax.experimental.pallas import tpu_sc as plsc

NUM_NODES = 1000000
DIM = 32

NC = 2
NS = 16
NW = NC * NS
COLS_PER_W = 31232
CHUNK = 1920
CHUNK_TAIL = 512
NCHUNK = 17
TAIL_BASE = COLS_PER_W * NW
TAIL0 = 512
TAIL1 = 64
NBUF = 2


def _chunk(g):
    return (g * CHUNK, CHUNK) if g < NCHUNK - 1 else ((NCHUNK - 1) * CHUNK, CHUNK_TAIL)


@functools.partial(
    pl.kernel,
    out_type=jax.ShapeDtypeStruct((DIM, NUM_NODES), jnp.float32),
    mesh=plsc.VectorSubcoreMesh(core_axis_name="c", subcore_axis_name="s"),
    scratch_types=[
        pltpu.VMEM((NBUF, DIM, CHUNK), jnp.float32),
        pltpu.VMEM((DIM, TAIL1), jnp.float32),
        pltpu.SemaphoreType.DMA,
        pltpu.SemaphoreType.DMA,
        pltpu.SemaphoreType.DMA,
        pltpu.SemaphoreType.DMA,
    ],
)
def _copy_table(wt_hbm, out_hbm, buf, tbuf, sem_in0, sem_in1, sem_out0, sem_out1):
    wid = lax.axis_index("s") * NC + lax.axis_index("c")
    base = pl.multiple_of(wid * COLS_PER_W, 128)
    sem_in = (sem_in0, sem_in1)
    sem_out = (sem_out0, sem_out1)

    def in_copy(g):
        b = g % NBUF
        off, w = _chunk(g)
        return pltpu.make_async_copy(
            wt_hbm.at[:, pl.ds(base + off, w)],
            buf.at[b, :, pl.ds(0, w)],
            sem_in[b],
        )

    def out_copy(g):
        b = g % NBUF
        off, w = _chunk(g)
        return pltpu.make_async_copy(
            buf.at[b, :, pl.ds(0, w)],
            out_hbm.at[:, pl.ds(base + off, w)],
            sem_out[b],
        )

    in_copy(0).start()
    in_copy(1).start()
    for g in range(NCHUNK):
        in_copy(g).wait()
        out_copy(g).start()
        if g + 2 < NCHUNK:
            out_copy(g).wait()
            in_copy(g + 2).start()
    out_copy(NCHUNK - 2).wait()
    out_copy(NCHUNK - 1).wait()

    @pl.when(wid == 0)
    def _tail():
        pltpu.sync_copy(
            wt_hbm.at[:, pl.ds(TAIL_BASE, TAIL0)], buf.at[0, :, pl.ds(0, TAIL0)]
        )
        pltpu.sync_copy(
            buf.at[0, :, pl.ds(0, TAIL0)], out_hbm.at[:, pl.ds(TAIL_BASE, TAIL0)]
        )
        pltpu.sync_copy(wt_hbm.at[:, pl.ds(TAIL_BASE + TAIL0, TAIL1)], tbuf)
        pltpu.sync_copy(tbuf, out_hbm.at[:, pl.ds(TAIL_BASE + TAIL0, TAIL1)])


def kernel(W):
    return _copy_table(W.T).T


# speedup vs baseline: 14.6488x; 1.0177x over previous
import functools

import jax
import jax.numpy as jnp
from jax import lax
from jax.experimental import pallas as pl
from jax.experimental.pallas import tpu as pltpu
from jax.experimental.pallas import tpu_sc as plsc

NUM_NODES = 1000000
DIM = 32

NC = 2
NS = 16
NW = NC * NS
COLS_PER_W = 31232
CHUNK = 1920
CHUNK_TAIL = 512
NCHUNK = 17
TAIL_BASE = COLS_PER_W * NW
TAIL0 = 512
TAIL1 = 64
NBUF = 2


def _chunk(g):
    return (g * CHUNK, CHUNK) if g < NCHUNK - 1 else ((NCHUNK - 1) * CHUNK, CHUNK_TAIL)


@functools.partial(
    pl.kernel,
    out_type=jax.ShapeDtypeStruct((DIM, NUM_NODES), jnp.float32),
    mesh=plsc.VectorSubcoreMesh(core_axis_name="c", subcore_axis_name="s"),
    scratch_types=[
        pltpu.VMEM((NBUF, DIM, CHUNK), jnp.float32),
        pltpu.VMEM((DIM, TAIL1), jnp.float32),
        pltpu.SemaphoreType.DMA,
        pltpu.SemaphoreType.DMA,
        pltpu.SemaphoreType.DMA,
        pltpu.SemaphoreType.DMA,
    ],
)
def _copy_table(wt_hbm, out_hbm, buf, tbuf, sem_in0, sem_in1, sem_out0, sem_out1):
    wid = lax.axis_index("s") * NC + lax.axis_index("c")
    base = pl.multiple_of(wid * COLS_PER_W, 128)
    sem_in = (sem_in0, sem_in1)
    sem_out = (sem_out0, sem_out1)

    def in_copy(g):
        b = g % NBUF
        off, w = _chunk(g)
        return pltpu.make_async_copy(
            wt_hbm.at[:, pl.ds(base + off, w)],
            buf.at[b, :, pl.ds(0, w)],
            sem_in[b],
        )

    def out_copy(g):
        b = g % NBUF
        off, w = _chunk(g)
        return pltpu.make_async_copy(
            buf.at[b, :, pl.ds(0, w)],
            out_hbm.at[:, pl.ds(base + off, w)],
            sem_out[b],
        )

    in_copy(0).start()
    in_copy(1).start()
    for g in range(NCHUNK):
        in_copy(g).wait()
        out_copy(g).start()
        if g + 2 < NCHUNK:
            out_copy(g).wait()
            in_copy(g + 2).start()
    out_copy(NCHUNK - 2).wait()
    out_copy(NCHUNK - 1).wait()

    @pl.when(wid == NW - 2)
    def _tail0():
        pltpu.sync_copy(
            wt_hbm.at[:, pl.ds(TAIL_BASE, TAIL0)], buf.at[0, :, pl.ds(0, TAIL0)]
        )
        pltpu.sync_copy(
            buf.at[0, :, pl.ds(0, TAIL0)], out_hbm.at[:, pl.ds(TAIL_BASE, TAIL0)]
        )

    @pl.when(wid == NW - 1)
    def _tail1():
        pltpu.sync_copy(wt_hbm.at[:, pl.ds(TAIL_BASE + TAIL0, TAIL1)], tbuf)
        pltpu.sync_copy(tbuf, out_hbm.at[:, pl.ds(TAIL_BASE + TAIL0, TAIL1)])


def kernel(W):
    return _copy_table(W.T).T
